# baseline (speedup 1.0000x reference)
import functools

import jax
import jax.numpy as jnp
import numpy as np
from jax import lax
from jax.experimental import pallas as pl
from jax.experimental.pallas import tpu as pltpu
from jax.experimental.pallas import tpu_sc as plsc

N_SAMPLES = 1024
N_VOCAB = 100000
LANES = 16
SG = 128
NG = 4
N_STRIPS = 4
STRIP_V = N_VOCAB // N_STRIPS
VJ = 8
TRJ = VJ // 8
N_FULL = STRIP_V // VJ
PAIRS = N_FULL // 2
TAIL_V = STRIP_V - N_FULL * VJ
KV = SG // LANES

_A0 = np.float32(2.885390081777927)
_A1 = np.float32(0.961796693925976)
_A2 = np.float32(0.577078016355585)
_A3 = np.float32(0.412198583111132)
_A4 = np.float32(0.320598897975325)
_KOFF = np.int32(0x3F800000 - 0x3F3504F3)
_MOFF = np.int32(0x3F3504F3)
_MMASK = np.int32(0x7FFFFF)
_ONE_BITS = np.int32(0x3F800000)
_THRC = np.float32(2.0 * np.log(2.0) * (1.0 + 3e-5) * 8388608.0)


def _neg_log2(u):
    bits = lax.bitcast_convert_type(u, jnp.int32)
    tmp = bits + _KOFF
    e = (tmp >> 23) - 127
    mb = (tmp & _MMASK) + _MOFF
    m = lax.bitcast_convert_type(mb, jnp.float32)
    s = m - 1.0
    t = s / (s + 2.0)
    z = t * t
    poly = t * (_A0 + z * (_A1 + z * (_A2 + z * (_A3 + z * _A4))))
    return -(e.astype(jnp.float32) + poly)


def _sc_body(param_hbm, noise_hbm, tasks_hbm, iw_hbm,
             wstrip, nbuf_a, nbuf_b, thr, bestv, besti, accbuf,
             msloc, mgloc, tbuf, ptask, iwbuf, shared,
             sem_a, sem_b, gsem):
    core = lax.axis_index("c")
    sid = lax.axis_index("s")
    strip = sid // NG
    group = sid % NG
    s_off = (core * NG + group) * SG
    v_off = strip * STRIP_V

    inf16 = jnp.full((LANES,), jnp.inf, jnp.float32)
    zero16 = jnp.zeros((LANES,), jnp.int32)
    iota16 = lax.iota(jnp.int32, LANES)
    lane_idx = [jnp.full((LANES,), jo, jnp.int32) for jo in range(LANES)]

    def lane_bcast(vec, jo):
        return jnp.full((LANES,), vec[jo], jnp.float32)

    for k in range(KV):
        thr[pl.ds(k * LANES, LANES)] = inf16
        bestv[pl.ds(k * LANES, LANES)] = inf16
        besti[pl.ds(k * LANES, LANES)] = zero16

    def nslice(c0, nv):
        return noise_hbm.at[pl.ds((v_off + c0) // 8, nv // 8), :,
                            pl.ds(s_off, SG)]

    pltpu.async_copy(nslice(0, VJ), nbuf_a, sem_a)

    pltpu.sync_copy(param_hbm.at[pl.ds(v_off, STRIP_V)],
                    wstrip.at[pl.ds(0, STRIP_V)])
    NW_FULL = STRIP_V // LANES

    def wexp_body(k, carry):
        m_run, s_run = carry
        pv = wstrip[pl.ds(k * LANES, LANES)]
        m_new = jnp.maximum(m_run, pv)
        s_run = s_run * jnp.exp(m_run - m_new) + jnp.exp(pv - m_new)
        wstrip[pl.ds(k * LANES, LANES)] = jnp.exp(-pv)
        return m_new, s_run
    m0 = jnp.full((LANES,), -jnp.inf, jnp.float32)
    s0 = jnp.zeros((LANES,), jnp.float32)
    m_run, s_run = lax.fori_loop(0, NW_FULL, wexp_body, (m0, s0))
    pv = wstrip[pl.ds(STRIP_V - LANES, LANES)]
    pm = jnp.where(iota16 >= 8, pv, -jnp.inf)
    m_new = jnp.maximum(m_run, pm)
    s_run = s_run * jnp.exp(m_run - m_new) + jnp.exp(pm - m_new)
    m_run = m_new
    wstrip[pl.ds(STRIP_V - LANES, LANES)] = jnp.where(
        iota16 >= 8, jnp.exp(-pv), pv)
    msloc[pl.ds(0, LANES)] = m_run
    msloc[pl.ds(LANES, LANES)] = s_run
    pltpu.sync_copy(msloc, shared.at[pl.ds(sid * 2 * LANES, 2 * LANES)])

    def scan_chunk(nb, base, ntr):
        def tr_body(tr, accs):
            wvec = wstrip[pl.ds(base + tr * 8, LANES)]
            accs = list(accs)
            for jo in range(8):
                bw = lane_bcast(wvec, jo)
                for k in range(KV):
                    u = nb[tr, jo, pl.ds(k * LANES, LANES)]
                    d = _ONE_BITS - lax.bitcast_convert_type(u, jnp.int32)
                    va = d.astype(jnp.float32) * bw
                    accs[k] = jnp.minimum(accs[k], va)
            return tuple(accs)
        accs = lax.fori_loop(0, ntr, tr_body, tuple([inf16] * KV))

        def precise_tr(pv, pi, ko, tr):
            wvec = wstrip[pl.ds(base + tr * 8, LANES)]
            for jo in range(8):
                bw = lane_bcast(wvec, jo)
                u = nb[tr, jo, pl.ds(ko, LANES)]
                v = _neg_log2(u) * bw
                jg = jnp.full((LANES,), v_off + base + tr * 8 + jo,
                              jnp.int32)
                ltm = v < pv
                pv = jnp.where(ltm, v, pv)
                pi = jnp.where(ltm, jg, pi)
            return pv, pi

        for k in range(KV):
            accbuf[pl.ds(k * LANES, LANES)] = accs[k]

        def trig_body(k, _):
            ko = k * LANES
            hit = accbuf[pl.ds(ko, LANES)] < thr[pl.ds(ko, LANES)]
            cnt = plsc.all_reduce_population_count(hit)

            @pl.when(cnt[0] > 0)
            def _():
                pv = bestv[pl.ds(ko, LANES)]
                pi = besti[pl.ds(ko, LANES)]

                def pj_body(tr, carry):
                    pv, pi = carry
                    return precise_tr(pv, pi, ko, tr)
                pv, pi = lax.fori_loop(0, ntr, pj_body, (pv, pi))
                bestv[pl.ds(ko, LANES)] = pv
                besti[pl.ds(ko, LANES)] = pi
                thr[pl.ds(ko, LANES)] = pv * _THRC
            return 0
        lax.fori_loop(0, KV, trig_body, 0)

    def pair_body(i, _):
        c_a = i * (2 * VJ)
        c_b = c_a + VJ
        pltpu.async_copy(nslice(c_b, VJ), nbuf_b, sem_b)
        pltpu.make_async_copy(nslice(c_a, VJ), nbuf_a, sem_a).wait()
        scan_chunk(nbuf_a, c_a, TRJ)

        @pl.when(i * 2 + 2 < N_FULL)
        def _():
            pltpu.async_copy(nslice(c_a + 2 * VJ, VJ), nbuf_a, sem_a)
        pltpu.make_async_copy(nslice(c_b, VJ), nbuf_b, sem_b).wait()
        scan_chunk(nbuf_b, c_b, TRJ)
        return 0
    lax.fori_loop(0, PAIRS, pair_body, 0)

    c_last = (N_FULL - 1) * VJ
    pltpu.make_async_copy(nslice(c_last, VJ), nbuf_a, sem_a).wait()
    scan_chunk(nbuf_a, c_last, TRJ)

    for k in range(KV):
        mgloc[pl.ds(k * LANES, LANES)] = bestv[pl.ds(k * LANES, LANES)]
        mgloc[pl.ds(SG + k * LANES, LANES)] = lax.bitcast_convert_type(
            besti[pl.ds(k * LANES, LANES)], jnp.float32)
    pltpu.sync_copy(mgloc,
                    shared.at[pl.ds(16 * 2 * LANES + sid * 2 * SG, 2 * SG)])
    plsc.subcore_barrier()

    @pl.when(sid < NG)
    def _():
        g = sid
        M = jnp.full((LANES,), -jnp.inf, jnp.float32)
        S = jnp.zeros((LANES,), jnp.float32)
        for q in range(N_STRIPS):
            pltpu.sync_copy(
                shared.at[pl.ds(q * NG * 2 * LANES, 2 * LANES)], msloc)
            mq = msloc[pl.ds(0, LANES)]
            sq = msloc[pl.ds(LANES, LANES)]
            Mn = jnp.maximum(M, mq)
            S = S * jnp.exp(M - Mn) + sq * jnp.exp(mq - Mn)
            M = Mn
        pmax = jnp.max(M)
        ssum = jnp.sum(S * jnp.exp(M - pmax))
        scale = ssum * np.float32(1.0 / N_VOCAB)

        for q in range(N_STRIPS):
            src = q * NG + g
            pltpu.sync_copy(
                shared.at[pl.ds(16 * 2 * LANES + src * 2 * SG, 2 * SG)],
                mgloc)
            for k in range(KV):
                qv = mgloc[pl.ds(k * LANES, LANES)]
                qi = lax.bitcast_convert_type(
                    mgloc[pl.ds(SG + k * LANES, LANES)], jnp.int32)
                if q == 0:
                    bestv[pl.ds(k * LANES, LANES)] = qv
                    besti[pl.ds(k * LANES, LANES)] = qi
                else:
                    cv = bestv[pl.ds(k * LANES, LANES)]
                    ci = besti[pl.ds(k * LANES, LANES)]
                    ltm = qv < cv
                    bestv[pl.ds(k * LANES, LANES)] = jnp.where(ltm, qv, cv)
                    besti[pl.ds(k * LANES, LANES)] = jnp.where(ltm, qi, ci)
        for k in range(KV):
            tbuf[pl.ds(k * LANES, LANES)] = besti[pl.ds(k * LANES, LANES)]

        pltpu.async_copy(param_hbm.at[tbuf], ptask, gsem).wait()
        for k in range(KV):
            pt = ptask[pl.ds(k * LANES, LANES)]
            iwbuf[pl.ds(k * LANES, LANES)] = jnp.exp(pmax - pt) * scale

        out0 = (core * NG + g) * SG
        pltpu.sync_copy(tbuf, tasks_hbm.at[pl.ds(out0, SG)])
        pltpu.sync_copy(iwbuf, iw_hbm.at[pl.ds(out0, SG)])


@jax.jit
def _run(parameter, noise_t):
    mesh = plsc.VectorSubcoreMesh(core_axis_name="c", subcore_axis_name="s")
    f = functools.partial(
        pl.kernel,
        out_type=(
            jax.ShapeDtypeStruct((N_SAMPLES,), jnp.int32),
            jax.ShapeDtypeStruct((N_SAMPLES,), jnp.float32),
        ),
        mesh=mesh,
        compiler_params=pltpu.CompilerParams(needs_layout_passes=False),
        scratch_types=[
            pltpu.VMEM((STRIP_V + 8,), jnp.float32),
            pltpu.VMEM((TRJ, 8, SG), jnp.float32),
            pltpu.VMEM((TRJ, 8, SG), jnp.float32),
            pltpu.VMEM((SG,), jnp.float32),
            pltpu.VMEM((SG,), jnp.float32),
            pltpu.VMEM((SG,), jnp.int32),
            pltpu.VMEM((SG,), jnp.float32),
            pltpu.VMEM((2 * LANES,), jnp.float32),
            pltpu.VMEM((2 * SG,), jnp.float32),
            pltpu.VMEM((SG,), jnp.int32),
            pltpu.VMEM((SG,), jnp.float32),
            pltpu.VMEM((SG,), jnp.float32),
            pltpu.VMEM_SHARED((16 * 2 * LANES + 16 * 2 * SG,),
                              jnp.float32),
            pltpu.SemaphoreType.DMA,
            pltpu.SemaphoreType.DMA,
            pltpu.SemaphoreType.DMA,
        ],
    )(_sc_body)
    return f(parameter, noise_t)


def kernel(parameter, noise, num_outputs):
    noise_3d = noise.T.reshape(N_VOCAB // 8, 8, N_SAMPLES)
    tasks, iw = _run(parameter, noise_3d)
    tasks = tasks + (num_outputs - num_outputs)
    return tasks, iw


# speedup vs baseline: 2.3421x; 2.3421x over previous
import functools

import jax
import jax.numpy as jnp
import numpy as np
from jax import lax
from jax.experimental import pallas as pl
from jax.experimental.pallas import tpu as pltpu
from jax.experimental.pallas import tpu_sc as plsc

N_SAMPLES = 1024
N_VOCAB = 100000
LANES = 16
SG = 128
NG = 4
N_STRIPS = 4
STRIP_V = N_VOCAB // N_STRIPS
VJ = 40
TRJ = VJ // 8
N_FULL = STRIP_V // VJ
PAIRS = N_FULL // 2
TAIL_V = STRIP_V - N_FULL * VJ
KV = SG // LANES

_A0 = np.float32(2.885390081777927)
_A1 = np.float32(0.961796693925976)
_A2 = np.float32(0.577078016355585)
_A3 = np.float32(0.412198583111132)
_A4 = np.float32(0.320598897975325)
_KOFF = np.int32(0x3F800000 - 0x3F3504F3)
_MOFF = np.int32(0x3F3504F3)
_MMASK = np.int32(0x7FFFFF)
_ONE_BITS = np.int32(0x3F800000)
_THRC = np.float32(2.0 * np.log(2.0) * (1.0 + 3e-5) * 8388608.0)


def _neg_log2(u):
    bits = lax.bitcast_convert_type(u, jnp.int32)
    tmp = bits + _KOFF
    e = (tmp >> 23) - 127
    mb = (tmp & _MMASK) + _MOFF
    m = lax.bitcast_convert_type(mb, jnp.float32)
    s = m - 1.0
    t = s / (s + 2.0)
    z = t * t
    poly = t * (_A0 + z * (_A1 + z * (_A2 + z * (_A3 + z * _A4))))
    return -(e.astype(jnp.float32) + poly)


def _sc_body(param_hbm, noise_hbm, tasks_hbm, iw_hbm,
             wstrip, nbuf_a, nbuf_b, thr, bestv, besti, accbuf,
             msloc, mgloc, tbuf, ptask, iwbuf, shared,
             sem_a, sem_b, gsem):
    core = lax.axis_index("c")
    sid = lax.axis_index("s")
    strip = sid // NG
    group = sid % NG
    s_off = (core * NG + group) * SG
    v_off = strip * STRIP_V

    inf16 = jnp.full((LANES,), jnp.inf, jnp.float32)
    zero16 = jnp.zeros((LANES,), jnp.int32)
    iota16 = lax.iota(jnp.int32, LANES)
    lane_idx = [jnp.full((LANES,), jo, jnp.int32) for jo in range(LANES)]

    def lane_bcast(vec, jo):
        return jnp.full((LANES,), vec[jo], jnp.float32)

    for k in range(KV):
        thr[pl.ds(k * LANES, LANES)] = inf16
        bestv[pl.ds(k * LANES, LANES)] = inf16
        besti[pl.ds(k * LANES, LANES)] = zero16

    def nslice(c0, nv):
        return noise_hbm.at[pl.ds((v_off + c0) // 8, nv // 8), :,
                            pl.ds(s_off, SG)]

    pltpu.async_copy(nslice(0, VJ), nbuf_a, sem_a)

    pltpu.sync_copy(param_hbm.at[pl.ds(v_off, STRIP_V)],
                    wstrip.at[pl.ds(0, STRIP_V)])
    NW_FULL = STRIP_V // LANES

    def wexp_body(k, carry):
        m_run, s_run = carry
        pv = wstrip[pl.ds(k * LANES, LANES)]
        m_new = jnp.maximum(m_run, pv)
        s_run = s_run * jnp.exp(m_run - m_new) + jnp.exp(pv - m_new)
        wstrip[pl.ds(k * LANES, LANES)] = jnp.exp(-pv)
        return m_new, s_run
    m0 = jnp.full((LANES,), -jnp.inf, jnp.float32)
    s0 = jnp.zeros((LANES,), jnp.float32)
    m_run, s_run = lax.fori_loop(0, NW_FULL, wexp_body, (m0, s0))
    pv = wstrip[pl.ds(STRIP_V - LANES, LANES)]
    pm = jnp.where(iota16 >= 8, pv, -jnp.inf)
    m_new = jnp.maximum(m_run, pm)
    s_run = s_run * jnp.exp(m_run - m_new) + jnp.exp(pm - m_new)
    m_run = m_new
    wstrip[pl.ds(STRIP_V - LANES, LANES)] = jnp.where(
        iota16 >= 8, jnp.exp(-pv), pv)
    msloc[pl.ds(0, LANES)] = m_run
    msloc[pl.ds(LANES, LANES)] = s_run
    pltpu.sync_copy(msloc, shared.at[pl.ds(sid * 2 * LANES, 2 * LANES)])

    def scan_chunk(nb, base, ntr):
        def tr_body(tr, accs):
            wvec = wstrip[pl.ds(base + tr * 8, LANES)]
            accs = list(accs)
            for jo in range(8):
                bw = lane_bcast(wvec, jo)
                for k in range(KV):
                    u = nb[tr, jo, pl.ds(k * LANES, LANES)]
                    d = _ONE_BITS - lax.bitcast_convert_type(u, jnp.int32)
                    va = d.astype(jnp.float32) * bw
                    accs[k] = jnp.minimum(accs[k], va)
            return tuple(accs)
        accs = lax.fori_loop(0, ntr, tr_body, tuple([inf16] * KV))

        def precise_tr(pv, pi, ko, tr):
            wvec = wstrip[pl.ds(base + tr * 8, LANES)]
            for jo in range(8):
                bw = lane_bcast(wvec, jo)
                u = nb[tr, jo, pl.ds(ko, LANES)]
                v = _neg_log2(u) * bw
                jg = jnp.full((LANES,), v_off + base + tr * 8 + jo,
                              jnp.int32)
                ltm = v < pv
                pv = jnp.where(ltm, v, pv)
                pi = jnp.where(ltm, jg, pi)
            return pv, pi

        def run_triggers(accs):
            for k in range(KV):
                accbuf[pl.ds(k * LANES, LANES)] = accs[k]

            def trig_body(k, _):
                ko = k * LANES
                hit = accbuf[pl.ds(ko, LANES)] < thr[pl.ds(ko, LANES)]
                cnt = plsc.all_reduce_population_count(hit)

                @pl.when(cnt[0] > 0)
                def _():
                    pv = bestv[pl.ds(ko, LANES)]
                    pi = besti[pl.ds(ko, LANES)]

                    def pj_body(tr, carry):
                        pv, pi = carry
                        return precise_tr(pv, pi, ko, tr)
                    pv, pi = lax.fori_loop(0, ntr, pj_body, (pv, pi))
                    bestv[pl.ds(ko, LANES)] = pv
                    besti[pl.ds(ko, LANES)] = pi
                    thr[pl.ds(ko, LANES)] = pv * _THRC
                return 0
            lax.fori_loop(0, KV, trig_body, 0)

        anyhit = None
        for k in range(KV):
            h = accs[k] < thr[pl.ds(k * LANES, LANES)]
            anyhit = h if anyhit is None else (anyhit | h)
        anycnt = plsc.all_reduce_population_count(anyhit)

        @pl.when(anycnt[0] > 0)
        def _():
            run_triggers(accs)

    def pair_body(i, _):
        c_a = i * (2 * VJ)
        c_b = c_a + VJ
        pltpu.async_copy(nslice(c_b, VJ), nbuf_b, sem_b)
        pltpu.make_async_copy(nslice(c_a, VJ), nbuf_a, sem_a).wait()
        scan_chunk(nbuf_a, c_a, TRJ)

        @pl.when(i * 2 + 2 < N_FULL)
        def _():
            pltpu.async_copy(nslice(c_a + 2 * VJ, VJ), nbuf_a, sem_a)
        pltpu.make_async_copy(nslice(c_b, VJ), nbuf_b, sem_b).wait()
        scan_chunk(nbuf_b, c_b, TRJ)
        return 0
    lax.fori_loop(0, PAIRS, pair_body, 0)

    c_last = (N_FULL - 1) * VJ
    pltpu.make_async_copy(nslice(c_last, VJ), nbuf_a, sem_a).wait()
    scan_chunk(nbuf_a, c_last, TRJ)

    for k in range(KV):
        mgloc[pl.ds(k * LANES, LANES)] = bestv[pl.ds(k * LANES, LANES)]
        mgloc[pl.ds(SG + k * LANES, LANES)] = lax.bitcast_convert_type(
            besti[pl.ds(k * LANES, LANES)], jnp.float32)
    pltpu.sync_copy(mgloc,
                    shared.at[pl.ds(16 * 2 * LANES + sid * 2 * SG, 2 * SG)])
    plsc.subcore_barrier()

    @pl.when(sid < NG)
    def _():
        g = sid
        M = jnp.full((LANES,), -jnp.inf, jnp.float32)
        S = jnp.zeros((LANES,), jnp.float32)
        for q in range(N_STRIPS):
            pltpu.sync_copy(
                shared.at[pl.ds(q * NG * 2 * LANES, 2 * LANES)], msloc)
            mq = msloc[pl.ds(0, LANES)]
            sq = msloc[pl.ds(LANES, LANES)]
            Mn = jnp.maximum(M, mq)
            S = S * jnp.exp(M - Mn) + sq * jnp.exp(mq - Mn)
            M = Mn
        pmax = jnp.max(M)
        ssum = jnp.sum(S * jnp.exp(M - pmax))
        scale = ssum * np.float32(1.0 / N_VOCAB)

        for q in range(N_STRIPS):
            src = q * NG + g
            pltpu.sync_copy(
                shared.at[pl.ds(16 * 2 * LANES + src * 2 * SG, 2 * SG)],
                mgloc)
            for k in range(KV):
                qv = mgloc[pl.ds(k * LANES, LANES)]
                qi = lax.bitcast_convert_type(
                    mgloc[pl.ds(SG + k * LANES, LANES)], jnp.int32)
                if q == 0:
                    bestv[pl.ds(k * LANES, LANES)] = qv
                    besti[pl.ds(k * LANES, LANES)] = qi
                else:
                    cv = bestv[pl.ds(k * LANES, LANES)]
                    ci = besti[pl.ds(k * LANES, LANES)]
                    ltm = qv < cv
                    bestv[pl.ds(k * LANES, LANES)] = jnp.where(ltm, qv, cv)
                    besti[pl.ds(k * LANES, LANES)] = jnp.where(ltm, qi, ci)
        for k in range(KV):
            tbuf[pl.ds(k * LANES, LANES)] = besti[pl.ds(k * LANES, LANES)]

        pltpu.async_copy(param_hbm.at[tbuf], ptask, gsem).wait()
        for k in range(KV):
            pt = ptask[pl.ds(k * LANES, LANES)]
            iwbuf[pl.ds(k * LANES, LANES)] = jnp.exp(pmax - pt) * scale

        out0 = (core * NG + g) * SG
        pltpu.sync_copy(tbuf, tasks_hbm.at[pl.ds(out0, SG)])
        pltpu.sync_copy(iwbuf, iw_hbm.at[pl.ds(out0, SG)])


@jax.jit
def _run(parameter, noise_t):
    mesh = plsc.VectorSubcoreMesh(core_axis_name="c", subcore_axis_name="s")
    f = functools.partial(
        pl.kernel,
        out_type=(
            jax.ShapeDtypeStruct((N_SAMPLES,), jnp.int32),
            jax.ShapeDtypeStruct((N_SAMPLES,), jnp.float32),
        ),
        mesh=mesh,
        compiler_params=pltpu.CompilerParams(needs_layout_passes=False),
        scratch_types=[
            pltpu.VMEM((STRIP_V + 8,), jnp.float32),
            pltpu.VMEM((TRJ, 8, SG), jnp.float32),
            pltpu.VMEM((TRJ, 8, SG), jnp.float32),
            pltpu.VMEM((SG,), jnp.float32),
            pltpu.VMEM((SG,), jnp.float32),
            pltpu.VMEM((SG,), jnp.int32),
            pltpu.VMEM((SG,), jnp.float32),
            pltpu.VMEM((2 * LANES,), jnp.float32),
            pltpu.VMEM((2 * SG,), jnp.float32),
            pltpu.VMEM((SG,), jnp.int32),
            pltpu.VMEM((SG,), jnp.float32),
            pltpu.VMEM((SG,), jnp.float32),
            pltpu.VMEM_SHARED((16 * 2 * LANES + 16 * 2 * SG,),
                              jnp.float32),
            pltpu.SemaphoreType.DMA,
            pltpu.SemaphoreType.DMA,
            pltpu.SemaphoreType.DMA,
        ],
    )(_sc_body)
    return f(parameter, noise_t)


def kernel(parameter, noise, num_outputs):
    noise_3d = noise.T.reshape(N_VOCAB // 8, 8, N_SAMPLES)
    tasks, iw = _run(parameter, noise_3d)
    tasks = tasks + (num_outputs - num_outputs)
    return tasks, iw
